# Initial kernel scaffold; baseline (speedup 1.0000x reference)
import functools

import jax
import jax.numpy as jnp
from jax import lax
from jax.experimental import pallas as pl
from jax.experimental.pallas import tpu as pltpu
from jax.experimental.pallas import tpu_sc as plsc

NC, NS, LANES = 2, 16, 16
NW = NC * NS
B, S, E = 1024, 200, 64
N = B * S
PW = N // NW
IDX_W = 128
IDX_ROWS = PW // IDX_W
CH = 640
NCHUNK = PW // CH
CH_IDX = CH // IDX_W

_mesh = plsc.VectorSubcoreMesh(core_axis_name="c", subcore_axis_name="s")


@functools.partial(
    pl.kernel,
    out_type=jax.ShapeDtypeStruct((N, E), jnp.float32),
    mesh=_mesh,
    scratch_types=[
        pltpu.VMEM((IDX_ROWS, IDX_W), jnp.int32),
        pltpu.VMEM((IDX_ROWS, IDX_W), jnp.int32),
        pltpu.VMEM((CH, E), jnp.float32),
        pltpu.SemaphoreType.DMA,
    ],
)
def _sc_embed(seq_hbm, seg_hbm, tok_hbm, comb_hbm, out_hbm,
              tok_idx_v, cmb_idx_v, rows_v, sem):
    cid = lax.axis_index("c")
    sid = lax.axis_index("s")
    wid = sid * NC + cid
    ibase = wid * IDX_ROWS

    pltpu.sync_copy(seq_hbm.at[pl.ds(ibase, IDX_ROWS)], tok_idx_v)
    pltpu.sync_copy(seg_hbm.at[pl.ds(ibase, IDX_ROWS)], cmb_idx_v)

    lane = lax.iota(jnp.int32, LANES)

    def idx_body(r, _):
        g_row = (ibase + r) * IDX_W
        for j in range(IDX_W // LANES):
            g = lane + (g_row + j * LANES)
            pos = lax.rem(g, S)
            lab = cmb_idx_v[r, pl.ds(j * LANES, LANES)]
            cmb_idx_v[r, pl.ds(j * LANES, LANES)] = pos * 3 + lab
        return 0

    lax.fori_loop(0, IDX_ROWS, idx_body, 0)

    def chunk_body(c, _):
        ir0 = c * CH_IDX
        for r in range(CH_IDX):
            pltpu.async_copy(comb_hbm.at[cmb_idx_v.at[ir0 + r]],
                             rows_v.at[pl.ds(r * IDX_W, IDX_W)], sem).wait()
        for r in range(CH_IDX):
            pltpu.async_copy(tok_hbm.at[tok_idx_v.at[ir0 + r]],
                             rows_v.at[pl.ds(r * IDX_W, IDX_W)], sem,
                             add=True).wait()
        pltpu.sync_copy(rows_v, out_hbm.at[pl.ds(wid * PW + c * CH, CH)])
        return 0

    lax.fori_loop(0, NCHUNK, chunk_body, 0)


@jax.jit
def kernel(sequence, segment_label, token_table, segment_table, pe):
    comb = (pe[:, None, :] + segment_table[None, :, :]).reshape(S * 3, E)
    seq2 = sequence.reshape(N // IDX_W, IDX_W).astype(jnp.int32)
    seg2 = segment_label.reshape(N // IDX_W, IDX_W).astype(jnp.int32)
    out = _sc_embed(seq2, seg2, token_table, comb)
    return out.reshape(B, S, E)


# speedup vs baseline: 1.2310x; 1.2310x over previous
import functools

import jax
import jax.numpy as jnp
from jax import lax
from jax.experimental import pallas as pl
from jax.experimental.pallas import tpu as pltpu
from jax.experimental.pallas import tpu_sc as plsc

NC, NS, LANES = 2, 16, 16
NW = NC * NS
B, S, E = 1024, 200, 64
N = B * S
PW = N // NW
IDX_W = 128
IDX_ROWS = PW // IDX_W
CH = 640
NCHUNK = PW // CH
CH_IDX = CH // IDX_W

_mesh = plsc.VectorSubcoreMesh(core_axis_name="c", subcore_axis_name="s")


@functools.partial(
    pl.kernel,
    out_type=jax.ShapeDtypeStruct((N, E), jnp.float32),
    mesh=_mesh,
    scratch_types=[
        pltpu.VMEM((IDX_ROWS, IDX_W), jnp.int32),
        pltpu.VMEM((IDX_ROWS, IDX_W), jnp.int32),
        pltpu.VMEM((CH, E), jnp.float32),
        pltpu.SemaphoreType.DMA,
    ],
    compiler_params=pltpu.CompilerParams(use_tc_tiling_on_sc=False),
)
def _sc_embed(seq_hbm, seg_hbm, tok_hbm, comb_hbm, out_hbm,
              tok_idx_v, cmb_idx_v, rows_v, sem):
    cid = lax.axis_index("c")
    sid = lax.axis_index("s")
    wid = sid * NC + cid
    ibase = wid * IDX_ROWS

    pltpu.sync_copy(seq_hbm.at[wid], tok_idx_v)
    pltpu.sync_copy(seg_hbm.at[wid], cmb_idx_v)

    lane = lax.iota(jnp.int32, LANES)

    def idx_body(r, _):
        g_row = (ibase + r) * IDX_W
        for j in range(IDX_W // LANES):
            g = lane + (g_row + j * LANES)
            pos = lax.rem(g, S)
            lab = cmb_idx_v[r, pl.ds(j * LANES, LANES)]
            cmb_idx_v[r, pl.ds(j * LANES, LANES)] = pos * 3 + lab
        return 0

    lax.fori_loop(0, IDX_ROWS, idx_body, 0)

    def chunk_body(c, _):
        ir0 = c * CH_IDX
        for r in range(CH_IDX):
            pltpu.async_copy(comb_hbm.at[cmb_idx_v.at[ir0 + r]],
                             rows_v.at[pl.ds(r * IDX_W, IDX_W)], sem).wait()
        for r in range(CH_IDX):
            pltpu.async_copy(tok_hbm.at[tok_idx_v.at[ir0 + r]],
                             rows_v.at[pl.ds(r * IDX_W, IDX_W)], sem,
                             add=True).wait()
        pltpu.sync_copy(rows_v, out_hbm.at[pl.ds(wid * PW + c * CH, CH)])
        return 0

    lax.fori_loop(0, NCHUNK, chunk_body, 0)


@jax.jit
def kernel(sequence, segment_label, token_table, segment_table, pe):
    comb = (pe[:, None, :] + segment_table[None, :, :]).reshape(S * 3, E)
    seq2 = sequence.reshape(NW, IDX_ROWS, IDX_W).astype(jnp.int32)
    seg2 = segment_label.reshape(NW, IDX_ROWS, IDX_W).astype(jnp.int32)
    out = _sc_embed(seq2, seg2, token_table, comb)
    return out.reshape(B, S, E)


# speedup vs baseline: 1.2761x; 1.0366x over previous
import functools

import jax
import jax.numpy as jnp
from jax import lax
from jax.experimental import pallas as pl
from jax.experimental.pallas import tpu as pltpu
from jax.experimental.pallas import tpu_sc as plsc

NC, NS, LANES = 2, 16, 16
NW = NC * NS
B, S, E = 1024, 200, 64
N = B * S
PW = N // NW
IDX_W = 128
IDX_ROWS = PW // IDX_W
CH = 640
NCHUNK = PW // CH
CH_IDX = CH // IDX_W

_mesh = plsc.VectorSubcoreMesh(core_axis_name="c", subcore_axis_name="s")


@functools.partial(
    pl.kernel,
    out_type=jax.ShapeDtypeStruct((N, E), jnp.float32),
    mesh=_mesh,
    scratch_types=[
        pltpu.VMEM((IDX_ROWS, IDX_W), jnp.int32),
        pltpu.VMEM((IDX_ROWS, IDX_W), jnp.int32),
        pltpu.VMEM((2, CH, E), jnp.float32),
        pltpu.SemaphoreType.DMA,
        pltpu.SemaphoreType.DMA,
        pltpu.SemaphoreType.DMA,
    ],
    compiler_params=pltpu.CompilerParams(use_tc_tiling_on_sc=False),
)
def _sc_embed(seq_hbm, seg_hbm, tok_hbm, comb_hbm, out_hbm,
              tok_idx_v, cmb_idx_v, rows_v, sem_g1, sem_g2, sem_out):
    cid = lax.axis_index("c")
    sid = lax.axis_index("s")
    wid = sid * NC + cid
    ibase = wid * IDX_ROWS

    pltpu.sync_copy(seq_hbm.at[wid], tok_idx_v)
    pltpu.sync_copy(seg_hbm.at[wid], cmb_idx_v)

    lane = lax.iota(jnp.int32, LANES)

    def idx_body(r, _):
        g_row = (ibase + r) * IDX_W
        for j in range(IDX_W // LANES):
            g = lane + (g_row + j * LANES)
            pos = lax.rem(g, S)
            lab = cmb_idx_v[r, pl.ds(j * LANES, LANES)]
            cmb_idx_v[r, pl.ds(j * LANES, LANES)] = pos * 3 + lab
        return 0

    lax.fori_loop(0, IDX_ROWS, idx_body, 0)

    def fire_g1(c, p):
        ir0 = c * CH_IDX
        for r in range(CH_IDX):
            pltpu.async_copy(comb_hbm.at[cmb_idx_v.at[ir0 + r]],
                             rows_v.at[p, pl.ds(r * IDX_W, IDX_W)], sem_g1)

    def drain(sem):
        pltpu.make_async_copy(out_hbm.at[pl.ds(0, CH)], rows_v.at[0], sem).wait()

    fire_g1(0, 0)

    def chunk_body(c, _):
        p = lax.rem(c, 2)
        ir0 = c * CH_IDX
        drain(sem_g1)
        for r in range(CH_IDX):
            pltpu.async_copy(tok_hbm.at[tok_idx_v.at[ir0 + r]],
                             rows_v.at[p, pl.ds(r * IDX_W, IDX_W)], sem_g2,
                             add=True)

        @pl.when(c >= 1)
        def _():
            drain(sem_out)

        @pl.when(c <= NCHUNK - 2)
        def _():
            fire_g1(c + 1, 1 - p)

        drain(sem_g2)
        pltpu.async_copy(rows_v.at[p], out_hbm.at[pl.ds(wid * PW + c * CH, CH)],
                         sem_out)
        return 0

    lax.fori_loop(0, NCHUNK, chunk_body, 0)
    drain(sem_out)


@jax.jit
def kernel(sequence, segment_label, token_table, segment_table, pe):
    comb = (pe[:, None, :] + segment_table[None, :, :]).reshape(S * 3, E)
    seq2 = sequence.reshape(NW, IDX_ROWS, IDX_W).astype(jnp.int32)
    seg2 = segment_label.reshape(NW, IDX_ROWS, IDX_W).astype(jnp.int32)
    out = _sc_embed(seq2, seg2, token_table, comb)
    return out.reshape(B, S, E)
